# baseline (speedup 1.0000x reference)
import jax
import jax.numpy as jnp
from jax import lax
from jax.experimental import pallas as pl
from jax.experimental.pallas import tpu as pltpu

EPS = 0.1
N_ITERS = 50
K = 8.0

B, N, D = 4, 8192, 768
KSEL = 512
N_CHUNK = 1024
N_STEPS = N // N_CHUNK


def _scores_from_s(s):
    constant = EPS * jnp.log(K)
    inv_eps = jnp.float32(1.0 / EPS)

    def one_iter(_, a):
        u = jnp.minimum(s + a, 0.0) * inv_eps
        ssum = jnp.sum(jnp.exp(u), axis=-1, keepdims=True)
        return a + (constant - EPS * jnp.log(ssum))

    a = lax.fori_loop(0, N_ITERS, one_iter, jnp.zeros((B, 1), jnp.float32))
    t1 = s + a
    bfin = -jax.nn.relu(t1)
    return jnp.exp((t1 + bfin) / EPS)


def _cumsum_lanes(x):
    n = x.shape[1]
    sh = 1
    while sh < n:
        shifted = jnp.concatenate(
            [jnp.zeros((x.shape[0], sh), x.dtype), x[:, :n - sh]], axis=1)
        x = x + shifted
        sh *= 2
    return x


def _topk_indices(sc, idx_ref, work_ref, ceq_ref):
    sb = lax.bitcast_convert_type(sc, jnp.int32)
    iota_n = lax.broadcasted_iota(jnp.int32, (B, N), 1)

    cnt_pos = jnp.sum(jnp.where(sb > 0, 1, 0), axis=1, keepdims=True)

    def bisect_all(_):
        def bisect_step(_, carry):
            lo, hi = carry
            mid = (lo + hi) // 2
            cnt = jnp.sum(jnp.where(sb >= mid, 1, 0), axis=1, keepdims=True)
            ok = cnt >= KSEL
            return (jnp.where(ok, mid, lo), jnp.where(ok, hi, mid))

        lo0 = jnp.zeros((B, 1), jnp.int32)
        hi0 = jnp.full((B, 1), 0x3F800001, jnp.int32)
        lo, _ = lax.fori_loop(0, 31, bisect_step, (lo0, hi0))
        return lo

    t = lax.cond(
        jnp.all(cnt_pos < KSEL),
        lambda _: jnp.zeros((B, 1), jnp.int32),
        bisect_all,
        operand=0,
    )

    mask_gt = sb > t
    g = jnp.sum(jnp.where(mask_gt, 1, 0), axis=1, keepdims=True)

    work_ref[:, :] = jnp.where(mask_gt, sc, -1.0)
    max_g = jnp.max(g)
    slot_iota = lax.broadcasted_iota(jnp.int32, (B, KSEL), 1)

    def extract_step(p, gslot):
        w = work_ref[:, :]
        m = jnp.max(w, axis=1, keepdims=True)
        amin = jnp.min(jnp.where(w == m, iota_n, N), axis=1, keepdims=True)
        work_ref[:, :] = jnp.where(iota_n == amin, -1.0, w)
        return jnp.where(slot_iota == p, amin, gslot)

    gslot = lax.fori_loop(
        0, max_g, extract_step, jnp.zeros((B, KSEL), jnp.int32))

    ceq_ref[:, :] = _cumsum_lanes(jnp.where(sb == t, 1, 0))
    lim = slot_iota - g
    CHN = 512

    def nchunk_step(j, acc):
        cc = ceq_ref[:, pl.ds(j * CHN, CHN)]
        part = jnp.sum(
            jnp.where(cc[:, None, :] <= lim[:, :, None], 1, 0), axis=2)
        return acc + part

    pmax = jnp.max(jnp.sum(jnp.where(ceq_ref[:, :] <= KSEL, 1, 0), axis=1))
    nchunks = (pmax + CHN - 1) // CHN
    eqpos = lax.fori_loop(
        0, nchunks, nchunk_step, jnp.zeros((B, KSEL), jnp.int32))

    idx_ref[:, :] = jnp.where(slot_iota < g, gslot, eqpos)


def _fused_body(x_ref, rt_ref, idx_ref, s_scr, work_scr, ceq_scr):
    j = pl.program_id(0)
    x = x_ref[:, :, :].reshape(B * N_CHUNK, D)
    rt = rt_ref[0]
    sv = lax.dot_general(
        x, rt[:, None],
        dimension_numbers=(((1,), (0,)), ((), ())),
        preferred_element_type=jnp.float32,
        precision=lax.Precision.DEFAULT,
    )
    s_scr[:, pl.ds(j * N_CHUNK, N_CHUNK)] = sv.reshape(B, N_CHUNK)

    @pl.when(j == N_STEPS - 1)
    def _():
        scores = _scores_from_s(s_scr[:, :])
        _topk_indices(scores, idx_ref, work_scr, ceq_scr)


def _compute(x, rt):
    return pl.pallas_call(
        _fused_body,
        grid=(N_STEPS,),
        in_specs=[
            pl.BlockSpec((B, N_CHUNK, D), lambda j: (0, j, 0)),
            pl.BlockSpec((1, D), lambda j: (0, 0)),
        ],
        out_specs=pl.BlockSpec((B, KSEL), lambda j: (0, 0)),
        out_shape=jax.ShapeDtypeStruct((B, KSEL), jnp.int32),
        scratch_shapes=[
            pltpu.VMEM((B, N), jnp.float32),
            pltpu.VMEM((B, N), jnp.float32),
            pltpu.VMEM((B, N), jnp.int32),
        ],
    )(x, rt[None, :])


def kernel(x, routing_token, num_tokens):
    n = x.shape[-2]
    num_tokens = jnp.minimum(num_tokens, n)
    idx = _compute(x, routing_token)
    valid = jnp.arange(KSEL) < num_tokens
    sel_scores = jnp.broadcast_to(
        jnp.where(valid, 1.0, 0.0).astype(jnp.float32), (x.shape[0], KSEL))
    sel_idx = jnp.where(valid, idx, 0)
    return (sel_scores, sel_idx)
